# Initial kernel scaffold; baseline (speedup 1.0000x reference)
import jax
import jax.numpy as jnp
from jax.experimental import pallas as pl


def kernel(x_0, node_idx, hedge_idx, W1, W2, A0, c0, A1, c1):
    raise NotImplementedError("write your pallas kernel here")


# speedup vs baseline: 4.6325x; 4.6325x over previous
import functools

import jax
import jax.numpy as jnp
from jax import lax
from jax.experimental import pallas as pl
from jax.experimental.pallas import tpu as pltpu
from jax.experimental.pallas import tpu_sc as plsc

N_NODES = 10000
N_HEDGES = 5000
NNZ = 320000
D = 128
NCLS = 16

NH_PAD = 5120
NN_PAD = 10240

_C = 80
_N_TILES = 32
_NNZ_PER_TILE = NNZ // _N_TILES
_N_CHUNKS = _NNZ_PER_TILE // _C


def _static_chunks(n, c=128):
    out = []
    o = 0
    while o < n:
        out.append((o, min(c, n - o)))
        o += c
    return out


def _make_spmm(n_src, n_dst, width):
    rows_per_tile = n_dst // 16
    mesh = plsc.VectorSubcoreMesh(core_axis_name="c", subcore_axis_name="s")

    @functools.partial(
        pl.kernel,
        out_type=jax.ShapeDtypeStruct((2, n_dst, width), jnp.float32),
        mesh=mesh,
        scratch_types=[
            pltpu.VMEM((_C,), jnp.int32),
            pltpu.VMEM((_C,), jnp.int32),
            pltpu.VMEM((_C, width), jnp.float32),
            pltpu.VMEM((128, width), jnp.float32),
            pltpu.VMEM_SHARED((n_dst, width), jnp.float32),
            pltpu.SemaphoreType.DMA,
        ],
    )
    def spmm(src_hbm, sidx_hbm, didx_hbm, out_hbm,
             sidx_v, didx_v, rows_v, zero_v, acc_sh, sem):
        c = lax.axis_index("c")
        s = lax.axis_index("s")
        tile = c * 16 + s

        z16 = jnp.zeros((16,), jnp.float32)

        def zrow(i, _):
            for j in range(width // 16):
                zero_v[i, pl.ds(j * 16, 16)] = z16
            return 0

        lax.fori_loop(0, 128, zrow, 0)
        row0 = s * rows_per_tile
        for off, n in _static_chunks(rows_per_tile):
            pltpu.sync_copy(zero_v.at[pl.ds(0, n)],
                            acc_sh.at[pl.ds(row0 + off, n)])
        plsc.subcore_barrier()

        base0 = tile * _NNZ_PER_TILE

        def chunk(i, _):
            base = base0 + i * _C
            pltpu.sync_copy(sidx_hbm.at[pl.ds(base, _C)], sidx_v)
            pltpu.sync_copy(didx_hbm.at[pl.ds(base, _C)], didx_v)
            pltpu.async_copy(src_hbm.at[sidx_v], rows_v, sem).wait()
            pltpu.sync_copy(rows_v, acc_sh.at[didx_v], add=True)
            return 0

        lax.fori_loop(0, _N_CHUNKS, chunk, 0)
        plsc.subcore_barrier()

        for off, n in _static_chunks(rows_per_tile):
            pltpu.sync_copy(acc_sh.at[pl.ds(row0 + off, n)],
                            out_hbm.at[c, pl.ds(row0 + off, n)])

    return spmm


_spmm_n2h = _make_spmm(N_NODES, NH_PAD, D)
_spmm_h2n = _make_spmm(NH_PAD, NN_PAD, D)
_spmm_n2h_b = _spmm_n2h


def _tc_call(body, out_shapes, *args):
    return pl.pallas_call(
        body,
        out_shape=out_shapes,
    )(*args)


def _comb_mm(p_ref, w_ref, o_ref):
    x = p_ref[0] + p_ref[1]
    o_ref[...] = jnp.dot(x, w_ref[...], preferred_element_type=jnp.float32)


def _comb_add(p_ref, o_ref):
    o_ref[...] = p_ref[0] + p_ref[1]


def _heads(p_ref, w2_ref, a1_ref, c1_ref, he_ref, m2_ref):
    x1b = p_ref[0] + p_ref[1]
    he_ref[...] = (jnp.dot(x1b, a1_ref[...], preferred_element_type=jnp.float32)
                   + c1_ref[...])
    m2_ref[...] = jnp.dot(x1b, w2_ref[...], preferred_element_type=jnp.float32)


def _final(p_ref, a0_ref, c0_ref, o_ref):
    o_ref[...] = (jnp.dot(p_ref[0] + p_ref[1], a0_ref[...],
                          preferred_element_type=jnp.float32) + c0_ref[...])


def kernel(x_0, node_idx, hedge_idx, W1, W2, A0, c0, A1, c1):
    ni = node_idx.astype(jnp.int32)
    hi = hedge_idx.astype(jnp.int32)

    p1 = _spmm_n2h(x_0, ni, hi)
    m1 = _tc_call(_comb_mm, jax.ShapeDtypeStruct((NH_PAD, D), jnp.float32),
                  p1, W1)
    p2 = _spmm_h2n(m1, hi, ni)
    x0b = _tc_call(_comb_add, jax.ShapeDtypeStruct((NN_PAD, D), jnp.float32),
                   p2)
    p3 = _spmm_n2h_b(x0b, ni, hi)
    out_he_pad, m2 = _tc_call(
        _heads,
        (jax.ShapeDtypeStruct((NH_PAD, NCLS), jnp.float32),
         jax.ShapeDtypeStruct((NH_PAD, D), jnp.float32)),
        p3, W2, A1, c1.reshape(1, NCLS))
    p4 = _spmm_h2n(m2, hi, ni)
    out_0 = _tc_call(_final,
                     jax.ShapeDtypeStruct((NN_PAD, NCLS), jnp.float32),
                     p4, A0, c0.reshape(1, NCLS))
    return (out_0[:N_NODES], out_he_pad[:N_HEDGES])
